# Initial kernel scaffold; baseline (speedup 1.0000x reference)
import jax
import jax.numpy as jnp
from jax.experimental import pallas as pl


def kernel(pred_conf, pred_loc, priory_boxes, truth):
    raise NotImplementedError("write your pallas kernel here")


# speedup vs baseline: 16.5833x; 16.5833x over previous
import functools

import jax
import jax.numpy as jnp
from jax import lax
from jax.experimental import pallas as pl
from jax.experimental.pallas import tpu as pltpu
from jax.experimental.pallas import tpu_sc as plsc

B = 32
P = 8732
C = 21
CPAD = 24
P2 = 8832
NEGPOS_RATIO = 3
THRESHHOLD = 0.1
NGT = 16

_SC_PARAMS = pltpu.CompilerParams(needs_layout_passes=False)


def _match_body(priors_ref, truth_ref, ploc_ref, pconf_ref,
                conf_ref, npos_ref, lloc_ref, gmax_ref):
    b = pl.program_id(0)
    lane = lax.broadcasted_iota(jnp.int32, (1, P2), 1).astype(jnp.float32)
    in_range = lane < float(P)

    pr = priors_ref[...]
    pcx, pcy, pw, ph = pr[0:1], pr[1:2], pr[2:3], pr[3:4]
    px1 = pcx - 0.5 * pw
    py1 = pcy - 0.5 * ph
    px2 = pcx + 0.5 * pw
    py2 = pcy + 0.5 * ph
    size_p = (px2 - px1) * (py2 - py1)

    t = truth_ref[0]
    tx1, ty1 = t[:, 0:1], t[:, 1:2]
    tx2, ty2 = t[:, 2:3], t[:, 3:4]
    tlab = t[:, 4:5]
    size_g = (tx2 - tx1) * (ty2 - ty1)

    ix1 = jnp.maximum(px1, tx1)
    iy1 = jnp.maximum(py1, ty1)
    ix2 = jnp.minimum(px2, tx2)
    iy2 = jnp.minimum(py2, ty2)
    cw = jnp.maximum(ix2 - ix1, 0.0)
    chh = jnp.maximum(iy2 - iy1, 0.0)
    cross = cw * chh
    total = size_p + size_g
    iou = cross / total - cross
    iou = jnp.where(in_range, iou, -1e30)

    giota = lax.broadcasted_iota(jnp.int32, (NGT, 1), 0).astype(jnp.float32)

    best_gt_overlap = jnp.max(iou, axis=0, keepdims=True)
    eq_g = iou == best_gt_overlap
    bgi = jnp.min(jnp.where(eq_g, giota, 99.0), axis=0, keepdims=True)

    best_p_overlap = jnp.max(iou, axis=1, keepdims=True)
    eq_p = iou == best_p_overlap
    bpi = jnp.min(jnp.where(eq_p, lane, 1e9), axis=1, keepdims=True)

    hit = lane == bpi
    winner = jnp.max(jnp.where(hit, giota, -1.0), axis=0, keepdims=True)
    bgi = jnp.where(winner >= 0.0, winner, bgi)

    onehot = bgi == giota
    conf = jnp.sum(jnp.where(onehot, tlab + 1.0, 0.0), axis=0, keepdims=True)
    mx1 = jnp.sum(jnp.where(onehot, tx1, 0.0), axis=0, keepdims=True)
    my1 = jnp.sum(jnp.where(onehot, ty1, 0.0), axis=0, keepdims=True)
    mx2 = jnp.sum(jnp.where(onehot, tx2, 0.0), axis=0, keepdims=True)
    my2 = jnp.sum(jnp.where(onehot, ty2, 0.0), axis=0, keepdims=True)

    conf = jnp.where(best_gt_overlap < THRESHHOLD, 0.0, conf)
    conf = jnp.where(in_range, conf, 0.0)
    pos = conf > 0.0

    l_cx = ((mx1 + mx2) / 2.0 - pcx) / pw
    l_cy = ((my1 + my2) / 2.0 - pcy) / ph
    l_w = jnp.log((mx2 - mx1) / pw)
    l_h = jnp.log((my2 - my1) / ph)

    pd = ploc_ref[0]
    d0 = jnp.abs(pd[0:1] - l_cx)
    d1 = jnp.abs(pd[1:2] - l_cy)
    d2 = jnp.abs(pd[2:3] - l_w)
    d3 = jnp.abs(pd[3:4] - l_h)

    def smooth(dd):
        return jnp.where(dd < 1.0, 0.5 * dd * dd, dd - 0.5)

    ssum = smooth(d0) + smooth(d1) + smooth(d2) + smooth(d3)
    lloc_part = jnp.sum(jnp.where(pos, ssum, 0.0))
    npos_part = jnp.sum(jnp.where(pos, 1.0, 0.0))
    bmax = jnp.max(pconf_ref[0])

    conf_ref[0] = conf
    npos_ref[0, 0, 0] = npos_part

    @pl.when(b == 0)
    def _():
        lloc_ref[0, 0] = 0.0
        gmax_ref[0, 0] = -jnp.inf

    lloc_ref[0, 0] += lloc_part
    gmax_ref[0, 0] = jnp.maximum(gmax_ref[0, 0], bmax)


def _run_match(priors_t, truth, ploc_t, pconf_t):
    return pl.pallas_call(
        _match_body,
        grid=(B,),
        in_specs=[
            pl.BlockSpec((4, P2), lambda b: (0, 0)),
            pl.BlockSpec((1, NGT, 5), lambda b: (b, 0, 0)),
            pl.BlockSpec((1, 4, P2), lambda b: (b, 0, 0)),
            pl.BlockSpec((1, CPAD, P2), lambda b: (b, 0, 0)),
        ],
        out_specs=[
            pl.BlockSpec((1, 1, P2), lambda b: (b, 0, 0)),
            pl.BlockSpec((1, 1, 1), lambda b: (b, 0, 0),
                         memory_space=pltpu.SMEM),
            pl.BlockSpec((1, 1), lambda b: (0, 0),
                         memory_space=pltpu.SMEM),
            pl.BlockSpec((1, 1), lambda b: (0, 0),
                         memory_space=pltpu.SMEM),
        ],
        out_shape=[
            jax.ShapeDtypeStruct((B, 1, P2), jnp.float32),
            jax.ShapeDtypeStruct((B, 1, 1), jnp.float32),
            jax.ShapeDtypeStruct((1, 1), jnp.float32),
            jax.ShapeDtypeStruct((1, 1), jnp.float32),
        ],
        compiler_params=pltpu.CompilerParams(
            dimension_semantics=("arbitrary",)),
    )(priors_t, truth, ploc_t, pconf_t)


def _keys_body(pconf_ref, conf_ref, gmax_ref,
               keys_ref, ce2_ref, cepos_ref):
    b = pl.program_id(0)
    lane = lax.broadcasted_iota(jnp.int32, (1, P2), 1).astype(jnp.float32)
    in_range = lane < float(P)

    x = pconf_ref[0]
    xmax = gmax_ref[0, 0]
    e2 = jnp.exp(x - xmax)
    s2 = jnp.sum(e2, axis=0, keepdims=True)
    lse2 = jnp.log(s2) + xmax

    m_r = jnp.max(x, axis=0, keepdims=True)
    s_r = jnp.sum(jnp.exp(x - m_r), axis=0, keepdims=True)
    lse_row = jnp.log(s_r) + m_r

    conf = conf_ref[0]
    pos = conf > 0.0
    tgt = conf
    ciota = lax.broadcasted_iota(jnp.int32, (CPAD, 1), 0).astype(jnp.float32)
    sel = ciota == tgt
    picked = jnp.sum(jnp.where(sel, x, 0.0), axis=0, keepdims=True)

    key = lse2 - picked
    key = jnp.where(pos, 0.0, key)
    key = jnp.where(in_range, key, jnp.inf)

    ce = lse_row - picked
    ce2 = jnp.where(pos, 0.0, ce)
    ce2 = jnp.where(in_range, ce2, 0.0)
    cepos_part = jnp.sum(jnp.where(pos & in_range, ce, 0.0))

    keys_ref[0] = key
    ce2_ref[0] = ce2

    @pl.when(b == 0)
    def _():
        cepos_ref[0, 0] = 0.0

    cepos_ref[0, 0] += cepos_part


def _run_keys(pconf_t, conf, gmax):
    return pl.pallas_call(
        _keys_body,
        grid=(B,),
        in_specs=[
            pl.BlockSpec((1, CPAD, P2), lambda b: (b, 0, 0)),
            pl.BlockSpec((1, 1, P2), lambda b: (b, 0, 0)),
            pl.BlockSpec((1, 1), lambda b: (0, 0),
                         memory_space=pltpu.SMEM),
        ],
        out_specs=[
            pl.BlockSpec((1, 1, P2), lambda b: (b, 0, 0)),
            pl.BlockSpec((1, 1, P2), lambda b: (b, 0, 0)),
            pl.BlockSpec((1, 1), lambda b: (0, 0),
                         memory_space=pltpu.SMEM),
        ],
        out_shape=[
            jax.ShapeDtypeStruct((B, 1, P2), jnp.float32),
            jax.ShapeDtypeStruct((B, 1, P2), jnp.float32),
            jax.ShapeDtypeStruct((1, 1), jnp.float32),
        ],
        compiler_params=pltpu.CompilerParams(
            dimension_semantics=("arbitrary",)),
    )(pconf_t, conf, gmax)


_NV = P2 // 16
_NBUCKET = 2048
_NHV = _NBUCKET // 16


def _sc_sort_body(keys_hbm, sig_hbm, kf, k0, i0, k1, i1, hist):
    wid = lax.axis_index("s") * 2 + lax.axis_index("c")
    pltpu.sync_copy(keys_hbm.at[wid], kf)

    lane16 = lax.iota(jnp.int32, 16)
    sign = jnp.full((16,), jnp.int32(-2147483648))

    def init_body(v, _):
        kb = plsc.bitcast(kf[pl.ds(v * 16, 16)], jnp.int32)
        m = lax.shift_right_arithmetic(kb, 31)
        u = lax.bitwise_xor(kb, lax.bitwise_or(m, sign))
        k0[pl.ds(v * 16, 16)] = u
        i0[pl.ds(v * 16, 16)] = v * 16 + lane16
        return 0

    lax.fori_loop(0, _NV, init_body, 0)

    bufs = [(k0, i0), (k1, i1), (k0, i0), (k1, i1)]
    for p, shift in enumerate((0, 11, 22)):
        src_k, src_i = bufs[p]
        dst_k, dst_i = bufs[p + 1]

        def zero_body(h, _):
            hist[pl.ds(h * 16, 16)] = jnp.zeros((16,), jnp.int32)
            return 0

        lax.fori_loop(0, _NHV, zero_body, 0)

        def hist_body(v, _, src_k=src_k):
            k = src_k[pl.ds(v * 16, 16)]
            d = lax.bitwise_and(
                lax.shift_right_logical(k, shift), jnp.int32(_NBUCKET - 1))
            cnt, last = plsc.scan_count(d)
            plsc.addupdate_scatter(hist, [d], cnt, mask=last)
            return 0

        lax.fori_loop(0, _NV, hist_body, 0)

        def scan_body(h, run):
            v = hist[pl.ds(h * 16, 16)]
            cs = plsc.cumsum(v)
            hist[pl.ds(h * 16, 16)] = cs - v + run
            return run + jnp.sum(v)

        lax.fori_loop(0, _NHV, scan_body, jnp.int32(0))

        def perm_body(v, _, src_k=src_k, src_i=src_i,
                      dst_k=dst_k, dst_i=dst_i):
            k = src_k[pl.ds(v * 16, 16)]
            iv = src_i[pl.ds(v * 16, 16)]
            d = lax.bitwise_and(
                lax.shift_right_logical(k, shift), jnp.int32(_NBUCKET - 1))
            cnt, last = plsc.scan_count(d)
            base = plsc.load_gather(hist, [d])
            dest = base + cnt - 1
            plsc.store_scatter(dst_k, [dest], k)
            plsc.store_scatter(dst_i, [dest], iv)
            plsc.addupdate_scatter(hist, [d], cnt, mask=last)
            return 0

        lax.fori_loop(0, _NV, perm_body, 0)

    pltpu.sync_copy(i1, sig_hbm.at[wid])


def _run_sc_sort(keys):
    mesh = plsc.VectorSubcoreMesh(core_axis_name="c", subcore_axis_name="s")
    f = pl.kernel(
        _sc_sort_body,
        out_type=jax.ShapeDtypeStruct((B, P2), jnp.int32),
        mesh=mesh,
        scratch_types=[
            pltpu.VMEM((P2,), jnp.float32),
            pltpu.VMEM((P2,), jnp.int32),
            pltpu.VMEM((P2,), jnp.int32),
            pltpu.VMEM((P2,), jnp.int32),
            pltpu.VMEM((P2,), jnp.int32),
            pltpu.VMEM((_NBUCKET,), jnp.int32),
        ],
        compiler_params=_SC_PARAMS,
    )
    return f(keys)


def _neg_body(sig_ref, ce2_ref, npos_ref, t_ref):
    b = pl.program_id(0)
    npos = npos_ref[0, 0, 0]
    k = jnp.minimum(NEGPOS_RATIO * npos, float(P - 1))
    sig = sig_ref[0].astype(jnp.float32)
    neg = sig < k
    t_part = jnp.sum(jnp.where(neg, ce2_ref[0], 0.0))

    @pl.when(b == 0)
    def _():
        t_ref[0, 0] = 0.0

    t_ref[0, 0] += t_part


def _run_neg(sig, ce2, npos):
    return pl.pallas_call(
        _neg_body,
        grid=(B,),
        in_specs=[
            pl.BlockSpec((1, 1, P2), lambda b: (b, 0, 0)),
            pl.BlockSpec((1, 1, P2), lambda b: (b, 0, 0)),
            pl.BlockSpec((1, 1, 1), lambda b: (b, 0, 0),
                         memory_space=pltpu.SMEM),
        ],
        out_specs=pl.BlockSpec((1, 1), lambda b: (0, 0),
                               memory_space=pltpu.SMEM),
        out_shape=jax.ShapeDtypeStruct((1, 1), jnp.float32),
        compiler_params=pltpu.CompilerParams(
            dimension_semantics=("arbitrary",)),
    )(sig, ce2, npos)


def kernel(pred_conf, pred_loc, priory_boxes, truth):
    pconf_t = jnp.transpose(pred_conf, (0, 2, 1))
    pconf_t = jnp.pad(pconf_t, ((0, 0), (0, CPAD - C), (0, P2 - P)),
                      constant_values=-jnp.inf)
    ploc_t = jnp.transpose(pred_loc, (0, 2, 1))
    ploc_t = jnp.pad(ploc_t, ((0, 0), (0, 0), (0, P2 - P)))
    priors_t = jnp.pad(priory_boxes.T, ((0, 0), (0, P2 - P)))
    pad_wh = jnp.zeros((4, P2 - P), jnp.float32).at[2:].set(1.0)
    priors_t = priors_t.at[:, P:].set(pad_wh)

    conf, npos, lloc, gmax = _run_match(priors_t, truth, ploc_t, pconf_t)
    keys, ce2, cepos = _run_keys(pconf_t, conf, gmax)
    sig = _run_sc_sort(jnp.reshape(keys, (B, P2)))
    t_neg = _run_neg(jnp.reshape(sig, (B, 1, P2)), ce2, npos)

    n = jnp.sum(npos)
    loss_loc = lloc[0, 0] / n
    loss_c = (cepos[0, 0] + t_neg[0, 0]) / n
    return (loss_loc, loss_c)
